# baseline (speedup 1.0000x reference)
import functools

import jax
import jax.numpy as jnp
from jax import lax
from jax.experimental import pallas as pl
from jax.experimental.pallas import tpu as pltpu
from jax.experimental.pallas import tpu_sc as plsc

NC = 2
NS = 16
NW = NC * NS
L = 16

BATCH = 4096
HIST = 200
HIDDEN = 64
N_NODE = 1000000
NBLK = (N_NODE + 127) // 128
N_PAD = NBLK * 128
BH = BATCH // 128
UNITS = (HIST // 8) * BH


def _rsqrt_vec(s):
    i = lax.bitcast_convert_type(s, jnp.int32)
    i = jnp.int32(0x5F3759DF) - (i >> 1)
    y = lax.bitcast_convert_type(i, jnp.float32)
    for _ in range(2):
        y = y * (1.5 - 0.5 * s * y * y)
    return y


@functools.partial(
    pl.kernel,
    out_type=jax.ShapeDtypeStruct((N_PAD, 128), jnp.float32),
    mesh=plsc.VectorSubcoreMesh(
        core_axis_name="c", subcore_axis_name="s", num_cores=NC
    ),
    compiler_params=pltpu.CompilerParams(use_tc_tiling_on_sc=True, needs_layout_passes=False),
    scratch_types=[
        pltpu.VMEM((4, HIDDEN, 128), jnp.float32),
        pltpu.VMEM((2, 128, 128), jnp.float32),
        pltpu.SemaphoreType.DMA((4,)),
        pltpu.SemaphoreType.DMA((2,)),
    ],
)
def _table_repack(tt_hbm, out_hbm, tin, tout, isem, osem):
    wid = lax.axis_index("s") * NC + lax.axis_index("c")
    lanes = lax.iota(jnp.int32, L)

    def _in_start(c, b):
        col = pl.multiple_of(c * 128, 128)
        for fh in range(8):
            pltpu.make_async_copy(
                tt_hbm.at[pl.ds(fh * 8, 8), pl.ds(col, 128)],
                tin.at[b, pl.ds(fh * 8, 8)],
                isem.at[b],
            ).start()

    def _in_wait(b):
        pltpu.make_async_copy(
            tt_hbm.at[:, pl.ds(0, 128)], tin.at[b], isem.at[b]
        ).wait()

    def _out_copy(c, b2):
        row = pl.multiple_of(c * 128, 128)
        return pltpu.make_async_copy(
            tout.at[b2], out_hbm.at[pl.ds(row, 128)], osem.at[b2]
        )

    fzero = jnp.zeros((L,), jnp.int32)
    fq_idx = [lanes + fq * L for fq in range(4)]

    def _transpose(b, b2):
        for nq in range(8):
            for j in range(L):
                n = nq * L + j
                idx_nv = fzero + n
                vs = [
                    plsc.load_gather(tin.at[b], [fq_idx[fq], idx_nv])
                    for fq in range(4)
                ]
                for fq in range(4):
                    tout[b2, n, pl.ds(fq * L, L)] = vs[fq]

    nk = (NBLK + 4 * NW - 1) // (4 * NW)

    def _blk(j):
        return jnp.minimum(j * NW + wid, NBLK - 1)

    for j in range(3):
        _in_start(_blk(j), j)

    def block(k, carry):
        for b in range(4):
            j = 4 * k + b
            c = _blk(j)
            _in_wait(b)

            @pl.when((k > 0) | (b >= 2))
            def _():
                _out_copy(0, b % 2).wait()

            _transpose(b, b % 2)
            _out_copy(c, b % 2).start()
            _in_start(_blk(j + 3), (b + 3) % 4)
        return carry

    lax.fori_loop(0, nk, block, 0)
    for b in range(2):
        _out_copy(0, b).wait()
    for b in range(3):
        _in_wait(b)


@functools.partial(
    pl.kernel,
    out_type=jax.ShapeDtypeStruct((HIST, 8, BH, 8, 128), jnp.float32),
    mesh=plsc.VectorSubcoreMesh(
        core_axis_name="c", subcore_axis_name="s", num_cores=NC
    ),
    compiler_params=pltpu.CompilerParams(needs_layout_passes=False),
    scratch_types=[
        pltpu.VMEM((2, 8, 128), jnp.int32),
        pltpu.VMEM((2, 128, 128), jnp.float32),
        pltpu.VMEM((2, 8, 8, 128), jnp.float32),
        pltpu.SemaphoreType.DMA((2,)),
        pltpu.SemaphoreType.DMA((2,)),
        pltpu.SemaphoreType.DMA((2,)),
    ],
)
def _gather_norm(idxt_hbm, rows_hbm, out_hbm, idxv, gbuf, obuf, isem, gsem, osem):
    wid = lax.axis_index("s") * NC + lax.axis_index("c")
    lanes = lax.iota(jnp.int32, L)
    nu2 = (UNITS // NW + 1) // 2

    def _u(k2, ib):
        return jnp.minimum(k2 * 2 + ib, UNITS // NW - 1) * NW + wid

    def _icopy(u, ib):
        h8 = u // BH
        bh = u % BH
        return pltpu.make_async_copy(
            idxt_hbm.at[pl.ds(pl.multiple_of(h8 * 8, 8), 8),
                        pl.ds(pl.multiple_of(bh * 128, 128), 128)],
            idxv.at[ib],
            isem.at[ib],
        )

    def _gather(ib, hh, b):
        return pltpu.make_async_copy(
            rows_hbm.at[idxv.at[ib, hh]], gbuf.at[b], gsem.at[b]
        )

    def _put(h, bh, b):
        return pltpu.make_async_copy(
            obuf.at[b], out_hbm.at[h, :, bh], osem.at[b]
        )

    fzero = jnp.zeros((L,), jnp.int32)

    def _normalize(b):
        def qbody(q, cq):
            idx_b = lanes + q * L
            q16 = q * L
            acc = [jnp.zeros((L,), jnp.float32) for _ in range(4)]
            for f0 in range(0, HIDDEN, 8):
                vs = [
                    plsc.load_gather(gbuf.at[b], [idx_b, fzero + (f0 + j)])
                    for j in range(8)
                ]
                for j in range(8):
                    f = f0 + j
                    obuf[b, f // 8, f % 8, pl.ds(q16, L)] = vs[j]
                    acc[j % 4] = acc[j % 4] + vs[j] * vs[j]
            sc = _rsqrt_vec((acc[0] + acc[1]) + (acc[2] + acc[3]))
            for f0 in range(0, HIDDEN, 8):
                ws = [
                    obuf[b, (f0 + j) // 8, (f0 + j) % 8, pl.ds(q16, L)]
                    for j in range(8)
                ]
                for j in range(8):
                    f = f0 + j
                    obuf[b, f // 8, f % 8, pl.ds(q16, L)] = ws[j] * sc
            return cq

        lax.fori_loop(0, 8, qbody, 0)

    _icopy(_u(0, 0), 0).start()

    def unit2(k2, carry):
        for ib in range(2):
            u = _u(k2, ib)
            h8 = u // BH
            bh = u % BH
            nxt = _u(k2 + (1 if ib == 1 else 0), 1 - ib)
            _icopy(nxt, 1 - ib).start()
            _icopy(u, ib).wait()
            _gather(ib, 0, 0).start()

            def pair(p, c2):
                for b in range(2):
                    hh = p * 2 + b

                    @pl.when(hh + 1 < 8)
                    def _():
                        _gather(ib, hh + 1, 1 - b).start()

                    _gather(ib, hh, b).wait()

                    @pl.when((k2 > 0) | (ib > 0) | (p > 0))
                    def _():
                        _put(0, 0, b).wait()

                    _normalize(b)
                    _put(h8 * 8 + hh, bh, b).start()
                return c2

            lax.fori_loop(0, 4, pair, 0)
        return carry

    lax.fori_loop(0, nu2, unit2, 0)
    _icopy(_u(0, 0), 0).wait()
    for b in range(2):
        _put(0, 0, b).wait()


def kernel(inputs, embedding_weight):
    idxt = inputs.T.astype(jnp.int32)
    tablet = embedding_weight.T
    rows = _table_repack(tablet)
    out5 = _gather_norm(idxt, rows)
    return (
        out5.transpose(2, 4, 0, 1, 3)
        .reshape(BATCH, HIST, HIDDEN)
    )


# speedup vs baseline: 1.7389x; 1.7389x over previous
import functools

import jax
import jax.numpy as jnp
from jax import lax
from jax.experimental import pallas as pl
from jax.experimental.pallas import tpu as pltpu
from jax.experimental.pallas import tpu_sc as plsc

NC = 2
NS = 16
NW = NC * NS
L = 16

BATCH = 4096
HIST = 200
HIDDEN = 64
B = BATCH * HIST
C = 128
ROWS_PER_W = B // NW
NCHUNK = ROWS_PER_W // C
NBUF = 2


_GATHER_DNUMS = lax.GatherDimensionNumbers(
    offset_dims=(), collapsed_slice_dims=(0,), start_index_map=(0,)
)


def _lane_perm(v, perm):
    return lax.gather(
        v,
        perm[:, None],
        _GATHER_DNUMS,
        slice_sizes=(1,),
        mode=lax.GatherScatterMode.PROMISE_IN_BOUNDS,
    )


def _lane_sum(v):
    lanes = lax.iota(jnp.int32, L)
    for d in (8, 4, 2, 1):
        v = v + _lane_perm(v, lanes ^ d)
    return v


def _rsqrt_vec(s):
    i = lax.bitcast_convert_type(s, jnp.int32)
    i = jnp.int32(0x5F3759DF) - (i >> 1)
    y = lax.bitcast_convert_type(i, jnp.float32)
    for _ in range(2):
        y = y * (1.5 - 0.5 * s * y * y)
    return y


def _norm_row(inb, outb, b, r):
    v0 = inb[b, r, pl.ds(0, L)]
    v1 = inb[b, r, pl.ds(L, L)]
    v2 = inb[b, r, pl.ds(2 * L, L)]
    v3 = inb[b, r, pl.ds(3 * L, L)]
    ss = _lane_sum(v0 * v0 + v1 * v1 + v2 * v2 + v3 * v3)
    sc = _rsqrt_vec(ss)
    outb[b, r, pl.ds(0, L)] = v0 * sc
    outb[b, r, pl.ds(L, L)] = v1 * sc
    outb[b, r, pl.ds(2 * L, L)] = v2 * sc
    outb[b, r, pl.ds(3 * L, L)] = v3 * sc


@functools.partial(
    pl.kernel,
    out_type=jax.ShapeDtypeStruct((B, HIDDEN), jnp.float32),
    mesh=plsc.VectorSubcoreMesh(
        core_axis_name="c", subcore_axis_name="s", num_cores=NC
    ),
    compiler_params=pltpu.CompilerParams(use_tc_tiling_on_sc=False),
    scratch_types=[
        pltpu.VMEM((ROWS_PER_W,), jnp.int32),
        pltpu.VMEM((NBUF, C, HIDDEN), jnp.float32),
        pltpu.VMEM((NBUF, C, HIDDEN), jnp.float32),
        pltpu.SemaphoreType.DMA((NBUF,)),
        pltpu.SemaphoreType.DMA((NBUF,)),
    ],
)
def _emb_norm(idx_hbm, table_hbm, out_hbm, idx_v, inb, outb, gsem, osem):
    wid = lax.axis_index("s") * NC + lax.axis_index("c")
    base = wid * ROWS_PER_W

    pltpu.sync_copy(idx_hbm.at[pl.ds(base, ROWS_PER_W)], idx_v)

    def _gather(j, b):
        return pltpu.make_async_copy(
            table_hbm.at[idx_v.at[pl.ds(j * C, C)]], inb.at[b], gsem.at[b]
        )

    def _put(j, b):
        return pltpu.make_async_copy(
            outb.at[b], out_hbm.at[pl.ds(base + j * C, C)], osem.at[b]
        )

    for b in range(NBUF):
        _gather(b, b).start()

    def chunk_body(i, carry):
        for b in range(NBUF):
            j = i * NBUF + b
            _gather(j, b).wait()

            @pl.when(j >= NBUF)
            def _():
                _put(j - NBUF, b).wait()

            def rows_body(g, c):
                r0 = g * 8
                for rr in range(8):
                    _norm_row(inb, outb, b, r0 + rr)
                return c

            lax.fori_loop(0, C // 8, rows_body, 0)

            _put(j, b).start()

            @pl.when(j + NBUF < NCHUNK)
            def _():
                _gather(j + NBUF, b).start()
        return carry

    lax.fori_loop(0, NCHUNK // NBUF, chunk_body, 0)

    for b in range(NBUF):
        _put(0, b).wait()


def kernel(inputs, embedding_weight):
    idx = inputs.reshape(B).astype(jnp.int32)
    out = _emb_norm(idx, embedding_weight)
    return out.reshape(BATCH, HIST, HIDDEN)
